# baseline (speedup 1.0000x reference)
import functools
import jax
import jax.numpy as jnp
from jax import lax
from jax.experimental import pallas as pl
from jax.experimental.pallas import tpu as pltpu
from jax.experimental.pallas import tpu_sc as plsc

_V = 1000000
_D = 64
_B = 16384

_CHUNK_ROWS = 512
_N_CHUNKS = 1954
_TAIL_CHUNK = 1953
_TAIL_BASE = _TAIL_CHUNK * _CHUNK_ROWS
_TAIL_ROWS = _V - _TAIL_BASE

_L = 16
_N_IDX_VREGS = _B // _L
_DUMP_ROW = _B

_K_MAX = 62
_MB_ROWS = _N_IDX_VREGS + _K_MAX + 2
_SEG_ROWS = _MB_ROWS * _L
_PERM_ROWS = 32 * _SEG_ROWS
_B_PER_W2 = _B // 32


def _make_scan_kernel():
    info = plsc.get_sparse_core_info()
    NC, NS = info.num_cores, info.num_subcores
    assert NC * NS == 32

    mesh = plsc.VectorSubcoreMesh(core_axis_name="c", subcore_axis_name="s")

    @functools.partial(
        pl.kernel,
        mesh=mesh,
        out_type=jax.ShapeDtypeStruct((_PERM_ROWS * 128,), jnp.float32),
        scratch_types=[
            pltpu.VMEM((_B + _L,), jnp.int32),
            pltpu.VMEM((_MB_ROWS * _L,), jnp.int32),
            pltpu.VMEM((64,), jnp.int32),
            pltpu.VMEM((64,), jnp.int32),
            pltpu.VMEM((8 * (_K_MAX + 2),), jnp.int32),
            pltpu.VMEM((2, _D, _CHUNK_ROWS), jnp.float32),
            pltpu.VMEM((2, 2048), jnp.float32),
            pltpu.SemaphoreType.DMA,
            pltpu.SemaphoreType.DMA,
        ],
        compiler_params=pltpu.CompilerParams(needs_layout_passes=False),
    )
    def scan_kernel(idx_hbm, tableT_hbm, tail_hbm, perm_hbm, idx_ext, mb,
                    hist, cur, offs8, slab, outstage, slab_sem, out_sem):
        w = lax.axis_index("s") * NC + lax.axis_index("c")
        iota = lax.iota(jnp.int32, _L)
        w_vec = jnp.full((_L,), w, jnp.int32)

        last_k = _K_MAX - 1

        def slab_descs(k):
            chunk = w + 32 * k
            cond_norm = (k < last_k) | ((k == last_k) & (w == 0))
            cond_tail = (k == last_k) & (w == 1)

            def norm(cg):
                return pltpu.make_async_copy(
                    tableT_hbm.at[
                        pl.ds(cg * 8, 8),
                        pl.ds(chunk * _CHUNK_ROWS, _CHUNK_ROWS),
                    ],
                    slab.at[k % 2, pl.ds(cg * 8, 8)],
                    slab_sem,
                )

            tail = lambda: pltpu.make_async_copy(
                tail_hbm, slab.at[k % 2, :, pl.ds(0, 128)], slab_sem
            )
            return cond_norm, cond_tail, norm, tail

        def fire_slab(k):
            cond_norm, cond_tail, norm, tail = slab_descs(k)

            def fire_all():
                for cg in range(8):
                    norm(cg).start()

            pl.when(cond_norm)(fire_all)
            pl.when(cond_tail)(lambda: tail().start())

        def wait_slab(k):
            cond_norm, cond_tail, norm, tail = slab_descs(k)

            def wait_all():
                for cg in range(8):
                    norm(cg).wait()

            pl.when(cond_norm)(wait_all)
            pl.when(cond_tail)(lambda: tail().wait())

        fire_slab(0)
        fire_slab(1)

        pltpu.sync_copy(idx_hbm, idx_ext.at[pl.ds(0, _B)])
        idx_ext[pl.ds(_B, _L)] = jnp.zeros((_L,), jnp.int32)

        zeros = jnp.zeros((_L,), jnp.int32)
        for g in range(4):
            hist[pl.ds(g * _L, _L)] = zeros
        dummy = jnp.full((_L,), _DUMP_ROW, jnp.int32)

        def mb_init_body(v, _):
            mb[pl.ds(v * _L, _L)] = dummy
            return _

        lax.fori_loop(0, _MB_ROWS, mb_init_body, None, unroll=8)

        cal, _unused = plsc.scan_count(zeros)
        adj = cal[0]

        def hist_body(i, _):
            rv = idx_ext[pl.ds(i * _L, _L)]
            chunk = lax.shift_right_logical(rv, 9)
            m = (chunk & 31) == w_vec
            key = lax.shift_right_logical(chunk, 5)
            cntv, last = plsc.scan_count(key, m)
            ml = m & last
            old = plsc.load_gather(hist, [key], mask=ml)
            plsc.store_scatter(hist, [key], old + cntv + 1 - adj, mask=ml)
            return _

        lax.fori_loop(0, _N_IDX_VREGS, hist_body, None, unroll=4)

        carry = jnp.zeros((), jnp.int32)
        for g in range(4):
            h = hist[pl.ds(g * _L, _L)]
            cnt_v = lax.shift_right_logical(h + 15, 4)
            inc = plsc.cumsum(cnt_v)
            offs_v = inc - cnt_v + carry
            carry = carry + inc[15]
            plsc.store_scatter(offs8, [(iota + g * _L) * 8], offs_v)
            cur[pl.ds(g * _L, _L)] = offs_v * _L

        def place_body(i, _):
            rv = idx_ext[pl.ds(i * _L, _L)]
            chunk = lax.shift_right_logical(rv, 9)
            m = (chunk & 31) == w_vec
            key = lax.shift_right_logical(chunk, 5)
            rank, last = plsc.scan_count(key, m)
            c0 = plsc.load_gather(cur, [key], mask=m)
            pos = c0 + rank - adj
            bvec = jnp.full((_L,), i * _L, jnp.int32) + iota
            plsc.store_scatter(mb, [pos], bvec, mask=m)
            plsc.store_scatter(cur, [key], pos + 1, mask=m & last)
            return _

        lax.fori_loop(0, _N_IDX_VREGS, place_body, None, unroll=4)

        seg_base = w * _SEG_ROWS * 128

        def out_dma(v):
            return pltpu.make_async_copy(
                outstage.at[v & 1],
                perm_hbm.at[pl.ds(seg_base + v * _L * 128, _L * 128)],
                out_sem,
            )

        def chunk_body(k, _):
            wait_slab(k)
            chunk_s = w + 32 * k
            chunk_vec = jnp.full((_L,), chunk_s, jnp.int32)
            o2 = offs8[pl.ds(k * 8, _L)]
            vs = o2[0]
            ve = o2[8]
            slab_k = slab.at[k % 2]

            def match_body(v, _):
                half = jnp.full((_L,), v & 1, jnp.int32)
                b16 = mb[pl.ds(v * _L, _L)]
                rv = plsc.load_gather(idx_ext, [b16])
                m = lax.shift_right_logical(rv, 9) == chunk_vec
                rr = rv & 511
                for c in range(_D):
                    c_vec = jnp.full((_L,), c, jnp.int32)
                    val = plsc.load_gather(slab_k, [c_vec, rr], mask=m)
                    plsc.store_scatter(
                        outstage, [half, iota * 128 + c_vec], val
                    )
                out_dma(v).start()
                pl.when(v > vs)(lambda: out_dma(v - 1).wait())
                return _

            lax.fori_loop(vs, ve, match_body, None)
            pl.when(ve > vs)(lambda: out_dma(ve - 1).wait())

            fire_slab(k + 2)
            return _

        lax.fori_loop(0, _K_MAX, chunk_body, None)

    return scan_kernel


def _make_permute_kernel():
    info = plsc.get_sparse_core_info()
    NC, NS = info.num_cores, info.num_subcores
    assert NC * NS == 32

    mesh = plsc.VectorSubcoreMesh(core_axis_name="c", subcore_axis_name="s")
    NB = 1984

    @functools.partial(
        pl.kernel,
        mesh=mesh,
        out_type=jax.ShapeDtypeStruct((_D, _B), jnp.float32),
        scratch_types=[
            pltpu.VMEM((_B,), jnp.int32),
            pltpu.VMEM((NB,), jnp.int32),
            pltpu.VMEM((NB,), jnp.int32),
            pltpu.VMEM((_B_PER_W2,), jnp.int32),
            pltpu.VMEM((2, 128, 128), jnp.float32),
            pltpu.VMEM((2, _D, 128), jnp.float32),
            pltpu.SemaphoreType.DMA,
            pltpu.SemaphoreType.DMA,
        ],
        compiler_params=pltpu.CompilerParams(
            use_tc_tiling_on_sc=False, needs_layout_passes=False
        ),
    )
    def permute_kernel(idx_hbm, perm_hbm, out_hbm, idx_v, hist, cur, rowsrc,
                       gbuf, tbuf, gsem, wsem):
        w = lax.axis_index("s") * NC + lax.axis_index("c")
        iota = lax.iota(jnp.int32, _L)
        w_vec = jnp.full((_L,), w, jnp.int32)

        pltpu.sync_copy(idx_hbm, idx_v)

        zeros = jnp.zeros((_L,), jnp.int32)
        for g in range(NB // _L):
            hist[pl.ds(g * _L, _L)] = zeros

        cal, _unused = plsc.scan_count(zeros)
        adj = cal[0]
        all_true = zeros == zeros

        def hist_body(i, _):
            rv = idx_v[pl.ds(i * _L, _L)]
            key = lax.shift_right_logical(rv, 9)
            cntv, last = plsc.scan_count(key, all_true)
            old = plsc.load_gather(hist, [key], mask=last)
            plsc.store_scatter(hist, [key], old + cntv + 1 - adj, mask=last)
            return _

        lax.fori_loop(0, _N_IDX_VREGS, hist_body, None, unroll=4)

        acc0 = jnp.zeros((_L,), jnp.int32)
        acc1 = jnp.zeros((_L,), jnp.int32)
        seg0 = (iota & 31) * _SEG_ROWS
        seg1 = ((iota + _L) & 31) * _SEG_ROWS
        for k in range(NB // 32):
            h0 = hist[pl.ds(k * 32, _L)]
            h1 = hist[pl.ds(k * 32 + _L, _L)]
            cur[pl.ds(k * 32, _L)] = seg0 + acc0 * _L
            cur[pl.ds(k * 32 + _L, _L)] = seg1 + acc1 * _L
            acc0 = acc0 + lax.shift_right_logical(h0 + 15, 4)
            acc1 = acc1 + lax.shift_right_logical(h1 + 15, 4)

        def place_body(i, _):
            rv = idx_v[pl.ds(i * _L, _L)]
            key = lax.shift_right_logical(rv, 9)
            rank, last = plsc.scan_count(key, all_true)
            c0 = plsc.load_gather(cur, [key], mask=all_true)
            row = c0 + rank - adj
            bvec = jnp.full((_L,), i * _L, jnp.int32) + iota
            mine = lax.shift_right_logical(bvec, 9) == w_vec
            plsc.store_scatter(rowsrc, [bvec & 511], row, mask=mine)
            plsc.store_scatter(cur, [key], row + 1, mask=last)
            return _

        lax.fori_loop(0, _N_IDX_VREGS, place_body, None, unroll=4)

        def gather_dma(g):
            return pltpu.make_async_copy(
                perm_hbm.at[rowsrc.at[pl.ds(g * 128, 128)]],
                gbuf.at[g % 2],
                gsem,
            )

        def write_dma(g):
            return pltpu.make_async_copy(
                tbuf.at[g % 2],
                out_hbm.at[:, pl.ds(w * _B_PER_W2 + g * 128, 128)],
                wsem,
            )

        gather_dma(0).start()
        gather_dma(1).start()
        for g in range(4):
            gather_dma(g).wait()
            gsel = jnp.full((_L,), g % 2, jnp.int32)
            for c in range(_D):
                c_vec = jnp.full((_L,), c, jnp.int32)
                for j in range(8):
                    bv = iota + j * _L
                    val = plsc.load_gather(gbuf, [gsel, bv, c_vec])
                    tbuf[g % 2, c, pl.ds(j * _L, _L)] = val
            write_dma(g).start()
            write_dma(g).wait()
            if g + 2 < 4:
                gather_dma(g + 2).start()

    return permute_kernel


_scan = _make_scan_kernel()
_permute = _make_permute_kernel()


@jax.jit
def kernel(idx, latent_codes):
    idx32 = idx.astype(jnp.int32)
    tail = jnp.zeros((_D, 128), jnp.float32)
    tail = tail.at[:, :_TAIL_ROWS].set(latent_codes[_TAIL_BASE:].T)
    perm = _scan(idx32, latent_codes.T, tail)
    outT = _permute(idx32, perm.reshape(_PERM_ROWS, 128))
    return outT.T


# speedup vs baseline: 1.1277x; 1.1277x over previous
import functools
import jax
import jax.numpy as jnp
from jax import lax
from jax.experimental import pallas as pl
from jax.experimental.pallas import tpu as pltpu
from jax.experimental.pallas import tpu_sc as plsc

_V = 1000000
_D = 64
_B = 16384

_CHUNK_ROWS = 512
_N_CHUNKS = 1954
_TAIL_CHUNK = 1953
_TAIL_BASE = _TAIL_CHUNK * _CHUNK_ROWS
_TAIL_ROWS = _V - _TAIL_BASE

_L = 16
_N_IDX_VREGS = _B // _L
_DUMP_ROW = _B

_K_MAX = 62
_MB_ROWS = _N_IDX_VREGS + _K_MAX + 2
_SEG_ROWS = _MB_ROWS * _L
_PERM_ROWS = 32 * _SEG_ROWS
_B_PER_W2 = _B // 32


def _make_scan_kernel():
    info = plsc.get_sparse_core_info()
    NC, NS = info.num_cores, info.num_subcores
    assert NC * NS == 32

    mesh = plsc.VectorSubcoreMesh(core_axis_name="c", subcore_axis_name="s")

    @functools.partial(
        pl.kernel,
        mesh=mesh,
        out_type=jax.ShapeDtypeStruct((_PERM_ROWS * 128,), jnp.float32),
        scratch_types=[
            pltpu.VMEM((_B + _L,), jnp.int32),
            pltpu.VMEM((_MB_ROWS * _L,), jnp.int32),
            pltpu.VMEM((64,), jnp.int32),
            pltpu.VMEM((64,), jnp.int32),
            pltpu.VMEM((8 * (_K_MAX + 2),), jnp.int32),
            pltpu.VMEM((2, _D, _CHUNK_ROWS), jnp.float32),
            pltpu.VMEM((2, 2048), jnp.float32),
            pltpu.SemaphoreType.DMA,
            pltpu.SemaphoreType.DMA,
        ],
        compiler_params=pltpu.CompilerParams(needs_layout_passes=False),
    )
    def scan_kernel(idx_hbm, tableT_hbm, tail_hbm, perm_hbm, idx_ext, mb,
                    hist, cur, offs8, slab, outstage, slab_sem, out_sem):
        w = lax.axis_index("s") * NC + lax.axis_index("c")
        iota = lax.iota(jnp.int32, _L)
        w_vec = jnp.full((_L,), w, jnp.int32)

        last_k = _K_MAX - 1

        def slab_descs(k):
            chunk = w + 32 * k
            cond_norm = (k < last_k) | ((k == last_k) & (w == 0))
            cond_tail = (k == last_k) & (w == 1)

            def norm(cg):
                return pltpu.make_async_copy(
                    tableT_hbm.at[
                        pl.ds(cg * 8, 8),
                        pl.ds(chunk * _CHUNK_ROWS, _CHUNK_ROWS),
                    ],
                    slab.at[k % 2, pl.ds(cg * 8, 8)],
                    slab_sem,
                )

            tail = lambda: pltpu.make_async_copy(
                tail_hbm, slab.at[k % 2, :, pl.ds(0, 128)], slab_sem
            )
            return cond_norm, cond_tail, norm, tail

        def fire_slab(k):
            cond_norm, cond_tail, norm, tail = slab_descs(k)

            def fire_all():
                for cg in range(8):
                    norm(cg).start()

            pl.when(cond_norm)(fire_all)
            pl.when(cond_tail)(lambda: tail().start())

        def wait_slab(k):
            cond_norm, cond_tail, norm, tail = slab_descs(k)

            def wait_all():
                for cg in range(8):
                    norm(cg).wait()

            pl.when(cond_norm)(wait_all)
            pl.when(cond_tail)(lambda: tail().wait())

        fire_slab(0)
        fire_slab(1)

        pltpu.sync_copy(idx_hbm, idx_ext.at[pl.ds(0, _B)])
        idx_ext[pl.ds(_B, _L)] = jnp.zeros((_L,), jnp.int32)

        zeros = jnp.zeros((_L,), jnp.int32)
        for g in range(4):
            hist[pl.ds(g * _L, _L)] = zeros
        dummy = jnp.full((_L,), _DUMP_ROW, jnp.int32)

        def mb_init_body(v, _):
            mb[pl.ds(v * _L, _L)] = dummy
            return _

        lax.fori_loop(0, _MB_ROWS, mb_init_body, None, unroll=8)

        cal, _unused = plsc.scan_count(zeros)
        adj = cal[0]

        def hist_body(i, _):
            rv = idx_ext[pl.ds(i * _L, _L)]
            chunk = lax.shift_right_logical(rv, 9)
            m = (chunk & 31) == w_vec
            key = lax.shift_right_logical(chunk, 5)
            cntv, last = plsc.scan_count(key, m)
            ml = m & last
            old = plsc.load_gather(hist, [key], mask=ml)
            plsc.store_scatter(hist, [key], old + cntv + 1 - adj, mask=ml)
            return _

        lax.fori_loop(0, _N_IDX_VREGS, hist_body, None, unroll=4)

        carry = jnp.zeros((), jnp.int32)
        for g in range(4):
            h = hist[pl.ds(g * _L, _L)]
            cnt_v = lax.shift_right_logical(h + 15, 4)
            inc = plsc.cumsum(cnt_v)
            offs_v = inc - cnt_v + carry
            carry = carry + inc[15]
            plsc.store_scatter(offs8, [(iota + g * _L) * 8], offs_v)
            cur[pl.ds(g * _L, _L)] = offs_v * _L

        def place_body(i, _):
            rv = idx_ext[pl.ds(i * _L, _L)]
            chunk = lax.shift_right_logical(rv, 9)
            m = (chunk & 31) == w_vec
            key = lax.shift_right_logical(chunk, 5)
            rank, last = plsc.scan_count(key, m)
            c0 = plsc.load_gather(cur, [key], mask=m)
            pos = c0 + rank - adj
            bvec = jnp.full((_L,), i * _L, jnp.int32) + iota
            plsc.store_scatter(mb, [pos], bvec, mask=m)
            plsc.store_scatter(cur, [key], pos + 1, mask=m & last)
            return _

        lax.fori_loop(0, _N_IDX_VREGS, place_body, None, unroll=4)

        seg_base = w * _SEG_ROWS * 128

        def out_dma(v):
            return pltpu.make_async_copy(
                outstage.at[v & 1],
                perm_hbm.at[pl.ds(seg_base + v * _L * 128, _L * 128)],
                out_sem,
            )

        def chunk_body(k, _):
            wait_slab(k)
            chunk_s = w + 32 * k
            chunk_vec = jnp.full((_L,), chunk_s, jnp.int32)
            o2 = offs8[pl.ds(k * 8, _L)]
            vs = o2[0]
            ve = o2[8]
            slab_k = slab.at[k % 2]

            def match_body(v, _):
                half = jnp.full((_L,), v & 1, jnp.int32)
                b16 = mb[pl.ds(v * _L, _L)]
                rv = plsc.load_gather(idx_ext, [b16])
                m = lax.shift_right_logical(rv, 9) == chunk_vec
                rr = rv & 511
                for c in range(_D):
                    c_vec = jnp.full((_L,), c, jnp.int32)
                    val = plsc.load_gather(slab_k, [c_vec, rr], mask=m)
                    plsc.store_scatter(
                        outstage, [half, iota * 128 + c_vec], val
                    )
                out_dma(v).start()
                pl.when(v > vs)(lambda: out_dma(v - 1).wait())
                return _

            lax.fori_loop(vs, ve, match_body, None)
            pl.when(ve > vs)(lambda: out_dma(ve - 1).wait())

            fire_slab(k + 2)
            return _

        lax.fori_loop(0, _K_MAX, chunk_body, None)

    return scan_kernel


def _make_permute_kernel():
    info = plsc.get_sparse_core_info()
    NC, NS = info.num_cores, info.num_subcores
    assert NC * NS == 32

    mesh = plsc.VectorSubcoreMesh(core_axis_name="c", subcore_axis_name="s")
    NB = 1984

    @functools.partial(
        pl.kernel,
        mesh=mesh,
        out_type=jax.ShapeDtypeStruct((_B, _D), jnp.float32),
        scratch_types=[
            pltpu.VMEM((_B,), jnp.int32),
            pltpu.VMEM((NB,), jnp.int32),
            pltpu.VMEM((NB,), jnp.int32),
            pltpu.VMEM((_B_PER_W2,), jnp.int32),
            pltpu.VMEM((2, 128, 128), jnp.float32),
            pltpu.SemaphoreType.DMA,
            pltpu.SemaphoreType.DMA,
        ],
        compiler_params=pltpu.CompilerParams(
            use_tc_tiling_on_sc=False, needs_layout_passes=False
        ),
    )
    def permute_kernel(idx_hbm, perm_hbm, out_hbm, idx_v, hist, cur, rowsrc,
                       gbuf, gsem, wsem):
        w = lax.axis_index("s") * NC + lax.axis_index("c")
        iota = lax.iota(jnp.int32, _L)
        w_vec = jnp.full((_L,), w, jnp.int32)

        pltpu.sync_copy(idx_hbm, idx_v)

        zeros = jnp.zeros((_L,), jnp.int32)
        for g in range(NB // _L):
            hist[pl.ds(g * _L, _L)] = zeros

        cal, _unused = plsc.scan_count(zeros)
        adj = cal[0]
        all_true = zeros == zeros

        def hist_body(i, _):
            rv = idx_v[pl.ds(i * _L, _L)]
            key = lax.shift_right_logical(rv, 9)
            cntv, last = plsc.scan_count(key, all_true)
            old = plsc.load_gather(hist, [key], mask=last)
            plsc.store_scatter(hist, [key], old + cntv + 1 - adj, mask=last)
            return _

        lax.fori_loop(0, _N_IDX_VREGS, hist_body, None, unroll=4)

        acc0 = jnp.zeros((_L,), jnp.int32)
        acc1 = jnp.zeros((_L,), jnp.int32)
        seg0 = (iota & 31) * _SEG_ROWS
        seg1 = ((iota + _L) & 31) * _SEG_ROWS
        for k in range(NB // 32):
            h0 = hist[pl.ds(k * 32, _L)]
            h1 = hist[pl.ds(k * 32 + _L, _L)]
            cur[pl.ds(k * 32, _L)] = seg0 + acc0 * _L
            cur[pl.ds(k * 32 + _L, _L)] = seg1 + acc1 * _L
            acc0 = acc0 + lax.shift_right_logical(h0 + 15, 4)
            acc1 = acc1 + lax.shift_right_logical(h1 + 15, 4)

        def place_body(i, _):
            rv = idx_v[pl.ds(i * _L, _L)]
            key = lax.shift_right_logical(rv, 9)
            rank, last = plsc.scan_count(key, all_true)
            c0 = plsc.load_gather(cur, [key], mask=all_true)
            row = c0 + rank - adj
            bvec = jnp.full((_L,), i * _L, jnp.int32) + iota
            mine = lax.shift_right_logical(bvec, 9) == w_vec
            plsc.store_scatter(rowsrc, [bvec & 511], row, mask=mine)
            plsc.store_scatter(cur, [key], row + 1, mask=last)
            return _

        lax.fori_loop(0, _N_IDX_VREGS, place_body, None, unroll=4)

        def gather_dma(g):
            return pltpu.make_async_copy(
                perm_hbm.at[rowsrc.at[pl.ds(g * 128, 128)]],
                gbuf.at[g % 2],
                gsem,
            )

        def write_dma(g):
            return pltpu.make_async_copy(
                gbuf.at[g % 2, :, pl.ds(0, _D)],
                out_hbm.at[pl.ds(w * _B_PER_W2 + g * 128, 128)],
                wsem,
            )

        gather_dma(0).start()
        gather_dma(1).start()
        for g in range(4):
            gather_dma(g).wait()
            write_dma(g).start()
            write_dma(g).wait()
            if g + 2 < 4:
                gather_dma(g + 2).start()

    return permute_kernel


_scan = _make_scan_kernel()
_permute = _make_permute_kernel()


@jax.jit
def kernel(idx, latent_codes):
    idx32 = idx.astype(jnp.int32)
    tail = jnp.zeros((_D, 128), jnp.float32)
    tail = tail.at[:, :_TAIL_ROWS].set(latent_codes[_TAIL_BASE:].T)
    perm = _scan(idx32, latent_codes.T, tail)
    out = _permute(idx32, perm.reshape(_PERM_ROWS, 128))
    return out
